# baseline (speedup 1.0000x reference)
import functools
import math

import jax
import jax.numpy as jnp
from jax import lax
from jax.experimental import pallas as pl
from jax.experimental.pallas import tpu as pltpu
from jax.experimental.pallas import tpu_sc as plsc


def _make_sc_count(seq, sk):
    info = plsc.get_sparse_core_info()
    nc, ns, nl = info.num_cores, info.num_subcores, info.num_lanes
    nw = nc * ns
    rows_per_w = seq // nw
    nch = -(-sk // nl)
    padw = nch * nl
    mesh = plsc.VectorSubcoreMesh(core_axis_name="c", subcore_axis_name="s")

    @functools.partial(
        pl.kernel, mesh=mesh,
        out_type=jax.ShapeDtypeStruct((seq, seq), jnp.float32),
        compiler_params=pltpu.CompilerParams(needs_layout_passes=False),
        scratch_types=[
            pltpu.VMEM((rows_per_w * padw,), jnp.int32),
            pltpu.VMEM((seq,), jnp.float32),
        ],
    )
    def sc_count(idx_hbm, c_hbm, idx_v, row_v):
        wid = lax.axis_index("s") * nc + lax.axis_index("c")
        base = wid * rows_per_w
        pltpu.sync_copy(idx_hbm.at[pl.ds(base * padw, rows_per_w * padw)], idx_v)

        zeros16 = jnp.zeros((nl,), jnp.float32)
        ones16 = jnp.ones((nl,), jnp.float32)
        valid_last = sk - (nch - 1) * nl
        lanes = lax.iota(jnp.int32, nl)
        full_mask = lanes < nl
        tail_mask = lanes < valid_last

        def zero_body(i, carry):
            row_v[pl.ds(i * nl, nl)] = zeros16
            return carry

        lax.fori_loop(0, seq // nl, zero_body, 0)

        def row_body(r, carry):
            for ci in range(nch):
                iv = idx_v[pl.ds(r * padw + ci * nl, nl)]
                msk = full_mask if ci < nch - 1 else tail_mask
                plsc.addupdate_scatter(row_v, [iv], ones16, mask=msk)
            pltpu.sync_copy(row_v, c_hbm.at[base + r])
            for ci in range(nch):
                iv = idx_v[pl.ds(r * padw + ci * nl, nl)]
                msk = full_mask if ci < nch - 1 else tail_mask
                plsc.store_scatter(row_v, [iv], zeros16, mask=msk)
            return carry

        lax.fori_loop(0, rows_per_w, row_body, 0)

    def run(idx):
        pad = jnp.zeros((seq, padw - sk), jnp.int32)
        flat = jnp.concatenate([idx, pad], axis=1).reshape(seq * padw)
        return sc_count(flat)

    return run


def _qkv_kernel(x_ref, wq_ref, wk_ref, wv_ref, bq_ref, bk_ref, bv_ref,
                q_ref, k_ref, v_ref):
    x = x_ref[...]
    nt = (((1,), (1,)), ((), ()))
    q_ref[...] = lax.dot_general(x, wq_ref[...], nt,
                                 preferred_element_type=jnp.float32) + bq_ref[...]
    k_ref[...] = lax.dot_general(x, wk_ref[...], nt,
                                 preferred_element_type=jnp.float32) + bk_ref[...]
    v_ref[...] = lax.dot_general(x, wv_ref[...], nt,
                                 preferred_element_type=jnp.float32) + bv_ref[...]


def _mcol_kernel(q_ref, k_ref, c_ref, m_ref, *, seq, dh, inv_sk):
    nt = (((1,), (1,)), ((), ()))
    neg = jnp.float32(-1e30)
    ch = 512
    for part in range(2):
        lo = part * dh
        q = q_ref[:, lo:lo + dh]
        k = k_ref[:, lo:lo + dh]
        mx_acc = jnp.full((seq, 1), neg, jnp.float32)
        sum_acc = jnp.zeros((seq, 1), jnp.float32)
        for ci in range(seq // ch):
            kc = k[ci * ch:(ci + 1) * ch, :]
            sch = lax.dot_general(q, kc, nt, preferred_element_type=jnp.float32)
            cc = c_ref[:, ci * ch:(ci + 1) * ch]
            mx_acc = jnp.maximum(mx_acc, jnp.max(jnp.where(cc > 0, sch, neg),
                                                 axis=1, keepdims=True))
            sum_acc = sum_acc + jnp.sum(sch * cc, axis=1, keepdims=True)
        m_col = mx_acc - sum_acc * inv_sk
        m_ref[:, :, part:part + 1] = m_col[None, :, :]


def _select_kernel(m_ref, ji_ref, *, seq, n_heads, n_top):
    work = jnp.transpose(m_ref[...])
    iota = lax.broadcasted_iota(jnp.int32, (n_heads, seq), 1)
    neg = jnp.float32(-1e30)
    for i in range(n_top):
        vmax = jnp.max(work, axis=1, keepdims=True)
        j = jnp.min(jnp.where(work == vmax, iota, seq),
                    axis=1, keepdims=True)
        ji_ref[:, i:i + 1] = j
        work = jnp.where(iota == j, neg, work)


def _sattn_kernel(q_ref, k_ref, v_ref, ji_ref, wo_ref, bo_ref, o_ref,
                  qsel_ref, cr_ref, cf_ref, *, seq, dh, n_top, scale):
    prog = pl.program_id(0)
    nt = (((1,), (1,)), ((), ()))
    for part in range(2):
        lo = part * dh
        head = 2 * prog + part
        k = k_ref[:, lo:lo + dh]
        v = v_ref[:, lo:lo + dh]

        def gather(i, carry):
            j = ji_ref[head, i]
            qsel_ref[pl.ds(i, 1), :] = q_ref[pl.ds(j, 1), lo:lo + dh]
            return carry

        lax.fori_loop(0, n_top, gather, 0)

        sred = lax.dot_general(qsel_ref[...], k, nt,
                               preferred_element_type=jnp.float32)
        sc = sred * scale
        mx = jnp.max(sc, axis=1, keepdims=True)
        e = jnp.exp(sc - mx)
        attn = e / jnp.sum(e, axis=1, keepdims=True)
        cr_ref[...] = lax.dot_general(attn, v, (((1,), (0,)), ((), ())),
                                      preferred_element_type=jnp.float32)

        vmean = jnp.mean(v, axis=0, keepdims=True)
        cf_ref[:, lo:lo + dh] = jnp.broadcast_to(vmean, (seq, dh))

        def scatter(i, carry):
            j = ji_ref[head, i]
            cf_ref[pl.ds(j, 1), lo:lo + dh] = cr_ref[pl.ds(i, 1), :]
            return carry

        lax.fori_loop(0, n_top, scatter, 0)

    @pl.when(prog == 0)
    def _():
        o_ref[...] = jnp.broadcast_to(bo_ref[...], o_ref.shape)

    o_ref[...] = o_ref[...] + lax.dot_general(
        cf_ref[...], wo_ref[...], nt, preferred_element_type=jnp.float32)


def kernel(x, Wq, bq, Wk, bk, Wv, bv, Wo, bo, index_sample):
    B, L, D = x.shape
    H = 32
    dh = D // H
    sk = index_sample.shape[1]
    n_top = min(5 * int(math.log(L)), L)

    x2 = x.reshape(L, D)
    idx = index_sample.astype(jnp.int32)
    f32 = jnp.float32

    counts = _make_sc_count(L, sk)(idx)

    wt = 256
    q, k, v = pl.pallas_call(
        _qkv_kernel,
        grid=(D // wt,),
        in_specs=[
            pl.BlockSpec((L, D), lambda j: (0, 0)),
            pl.BlockSpec((wt, D), lambda j: (j, 0)),
            pl.BlockSpec((wt, D), lambda j: (j, 0)),
            pl.BlockSpec((wt, D), lambda j: (j, 0)),
            pl.BlockSpec((1, wt), lambda j: (0, j)),
            pl.BlockSpec((1, wt), lambda j: (0, j)),
            pl.BlockSpec((1, wt), lambda j: (0, j)),
        ],
        out_specs=[
            pl.BlockSpec((L, wt), lambda j: (0, j)),
            pl.BlockSpec((L, wt), lambda j: (0, j)),
            pl.BlockSpec((L, wt), lambda j: (0, j)),
        ],
        out_shape=[jax.ShapeDtypeStruct((L, D), f32)] * 3,
    )(x2, Wq, Wk, Wv, bq.reshape(1, D), bk.reshape(1, D), bv.reshape(1, D))

    m3 = pl.pallas_call(
        functools.partial(_mcol_kernel, seq=L, dh=dh, inv_sk=1.0 / sk),
        grid=(H // 2,),
        in_specs=[
            pl.BlockSpec((L, 2 * dh), lambda h: (0, h)),
            pl.BlockSpec((L, 2 * dh), lambda h: (0, h)),
            pl.BlockSpec((L, L), lambda h: (0, 0)),
        ],
        out_specs=pl.BlockSpec((1, L, 2), lambda h: (h, 0, 0)),
        out_shape=jax.ShapeDtypeStruct((H // 2, L, 2), f32),
    )(q, k, counts)
    m_all = m3.reshape(H // 2, L, 2).transpose(1, 0, 2).reshape(L, H)

    ji = pl.pallas_call(
        functools.partial(_select_kernel, seq=L, n_heads=H, n_top=n_top),
        in_specs=[pl.BlockSpec((L, H), lambda: (0, 0))],
        out_specs=pl.BlockSpec((H, n_top), lambda: (0, 0)),
        out_shape=jax.ShapeDtypeStruct((H, n_top), jnp.int32),
    )(m_all)

    out = pl.pallas_call(
        functools.partial(_sattn_kernel, seq=L, dh=dh, n_top=n_top,
                          scale=1.0 / math.sqrt(dh)),
        grid=(H // 2,),
        in_specs=[
            pl.BlockSpec((L, 2 * dh), lambda h: (0, h)),
            pl.BlockSpec((L, 2 * dh), lambda h: (0, h)),
            pl.BlockSpec((L, 2 * dh), lambda h: (0, h)),
            pl.BlockSpec(memory_space=pltpu.SMEM),
            pl.BlockSpec((D, 2 * dh), lambda h: (0, h)),
            pl.BlockSpec((1, D), lambda h: (0, 0)),
        ],
        out_specs=pl.BlockSpec((L, D), lambda h: (0, 0)),
        out_shape=jax.ShapeDtypeStruct((L, D), f32),
        scratch_shapes=[
            pltpu.VMEM((n_top, dh), f32),
            pltpu.VMEM((n_top, dh), f32),
            pltpu.VMEM((L, 2 * dh), f32),
        ],
    )(q, k, v, ji, Wo, bo.reshape(1, D))

    return out.reshape(B, L, D)


# speedup vs baseline: 1.1055x; 1.1055x over previous
import functools
import math

import jax
import jax.numpy as jnp
from jax import lax
from jax.experimental import pallas as pl
from jax.experimental.pallas import tpu as pltpu
from jax.experimental.pallas import tpu_sc as plsc


def _make_sc_count(seq, sk):
    info = plsc.get_sparse_core_info()
    nc, ns, nl = info.num_cores, info.num_subcores, info.num_lanes
    nw = nc * ns
    rows_per_w = seq // nw
    nch = -(-sk // nl)
    padw = nch * nl
    mesh = plsc.VectorSubcoreMesh(core_axis_name="c", subcore_axis_name="s")

    @functools.partial(
        pl.kernel, mesh=mesh,
        out_type=jax.ShapeDtypeStruct((seq, seq), jnp.float32),
        compiler_params=pltpu.CompilerParams(needs_layout_passes=False),
        scratch_types=[
            pltpu.VMEM((rows_per_w * padw,), jnp.int32),
            pltpu.VMEM((seq,), jnp.float32),
        ],
    )
    def sc_count(idx_hbm, c_hbm, idx_v, row_v):
        wid = lax.axis_index("s") * nc + lax.axis_index("c")
        base = wid * rows_per_w
        pltpu.sync_copy(idx_hbm.at[pl.ds(base * padw, rows_per_w * padw)], idx_v)

        zeros16 = jnp.zeros((nl,), jnp.float32)
        ones16 = jnp.ones((nl,), jnp.float32)
        valid_last = sk - (nch - 1) * nl
        lanes = lax.iota(jnp.int32, nl)
        full_mask = lanes < nl
        tail_mask = lanes < valid_last

        def zero_body(i, carry):
            row_v[pl.ds(i * nl, nl)] = zeros16
            return carry

        lax.fori_loop(0, seq // nl, zero_body, 0)

        def row_body(r, carry):
            for ci in range(nch):
                iv = idx_v[pl.ds(r * padw + ci * nl, nl)]
                msk = full_mask if ci < nch - 1 else tail_mask
                plsc.addupdate_scatter(row_v, [iv], ones16, mask=msk)
            pltpu.sync_copy(row_v, c_hbm.at[base + r])
            for ci in range(nch):
                iv = idx_v[pl.ds(r * padw + ci * nl, nl)]
                msk = full_mask if ci < nch - 1 else tail_mask
                plsc.store_scatter(row_v, [iv], zeros16, mask=msk)
            return carry

        lax.fori_loop(0, rows_per_w, row_body, 0)

    def run(idx):
        pad = jnp.zeros((seq, padw - sk), jnp.int32)
        flat = jnp.concatenate([idx, pad], axis=1).reshape(seq * padw)
        return sc_count(flat)

    return run


def _qkv_kernel(x_ref, wq_ref, wk_ref, wv_ref, bq_ref, bk_ref, bv_ref,
                q_ref, k_ref, v_ref):
    x = x_ref[...]
    nt = (((1,), (1,)), ((), ()))
    q_ref[...] = lax.dot_general(x, wq_ref[...], nt,
                                 preferred_element_type=jnp.float32) + bq_ref[...]
    k_ref[...] = lax.dot_general(x, wk_ref[...], nt,
                                 preferred_element_type=jnp.float32) + bk_ref[...]
    v_ref[...] = lax.dot_general(x, wv_ref[...], nt,
                                 preferred_element_type=jnp.float32) + bv_ref[...]


def _mcol_kernel(q_ref, k_ref, c_ref, m_ref, *, seq, dh, inv_sk):
    nt = (((1,), (1,)), ((), ()))
    neg = jnp.float32(-1e30)
    ch = 1024
    for part in range(2):
        lo = part * dh
        q = q_ref[:, lo:lo + dh]
        k = k_ref[:, lo:lo + dh]
        mx_acc = jnp.full((seq, 1), neg, jnp.float32)
        sum_acc = jnp.zeros((seq, 1), jnp.float32)
        for ci in range(seq // ch):
            kc = k[ci * ch:(ci + 1) * ch, :]
            sch = lax.dot_general(q, kc, nt, preferred_element_type=jnp.float32)
            cc = c_ref[:, ci * ch:(ci + 1) * ch]
            mx_acc = jnp.maximum(mx_acc, jnp.max(jnp.where(cc > 0, sch, neg),
                                                 axis=1, keepdims=True))
            sum_acc = sum_acc + jnp.sum(sch * cc, axis=1, keepdims=True)
        m_col = mx_acc - sum_acc * inv_sk
        m_ref[:, :, part:part + 1] = m_col[None, :, :]


def _select_kernel(m_ref, ji_ref, *, seq, n_heads, n_top):
    work = jnp.transpose(m_ref[...])
    iota = lax.broadcasted_iota(jnp.int32, (n_heads, seq), 1)
    neg = jnp.float32(-1e30)
    for i in range(n_top):
        vmax = jnp.max(work, axis=1, keepdims=True)
        j = jnp.min(jnp.where(work == vmax, iota, seq),
                    axis=1, keepdims=True)
        ji_ref[:, i:i + 1] = j
        work = jnp.where(iota == j, neg, work)


def _sattn_kernel(q_ref, k_ref, v_ref, ji_ref, o_ref, qsel_ref, cr_ref, *,
                  seq, dh, n_top, scale):
    prog = pl.program_id(0)
    nt = (((1,), (1,)), ((), ()))
    for part in range(2):
        lo = part * dh
        head = 2 * prog + part
        k = k_ref[:, lo:lo + dh]
        v = v_ref[:, lo:lo + dh]

        def gather(i, carry):
            j = ji_ref[head, i]
            qsel_ref[pl.ds(i, 1), :] = q_ref[pl.ds(j, 1), lo:lo + dh]
            return carry

        lax.fori_loop(0, n_top, gather, 0)

        sred = lax.dot_general(qsel_ref[...], k, nt,
                               preferred_element_type=jnp.float32)
        sc = sred * scale
        mx = jnp.max(sc, axis=1, keepdims=True)
        e = jnp.exp(sc - mx)
        attn = e / jnp.sum(e, axis=1, keepdims=True)
        cr_ref[...] = lax.dot_general(attn, v, (((1,), (0,)), ((), ())),
                                      preferred_element_type=jnp.float32)

        vmean = jnp.mean(v, axis=0, keepdims=True)
        o_ref[:, lo:lo + dh] = jnp.broadcast_to(vmean, (seq, dh))

        def scatter(i, carry):
            j = ji_ref[head, i]
            o_ref[pl.ds(j, 1), lo:lo + dh] = cr_ref[pl.ds(i, 1), :]
            return carry

        lax.fori_loop(0, n_top, scatter, 0)


def _out_kernel(x_ref, w_ref, b_ref, o_ref):
    o_ref[...] = lax.dot_general(x_ref[...], w_ref[...], (((1,), (1,)), ((), ())),
                                 preferred_element_type=jnp.float32) + b_ref[...]


def kernel(x, Wq, bq, Wk, bk, Wv, bv, Wo, bo, index_sample):
    B, L, D = x.shape
    H = 32
    dh = D // H
    sk = index_sample.shape[1]
    n_top = min(5 * int(math.log(L)), L)

    x2 = x.reshape(L, D)
    idx = index_sample.astype(jnp.int32)
    f32 = jnp.float32

    counts = _make_sc_count(L, sk)(idx)

    wt = 256
    q, k, v = pl.pallas_call(
        _qkv_kernel,
        grid=(D // wt,),
        in_specs=[
            pl.BlockSpec((L, D), lambda j: (0, 0)),
            pl.BlockSpec((wt, D), lambda j: (j, 0)),
            pl.BlockSpec((wt, D), lambda j: (j, 0)),
            pl.BlockSpec((wt, D), lambda j: (j, 0)),
            pl.BlockSpec((1, wt), lambda j: (0, j)),
            pl.BlockSpec((1, wt), lambda j: (0, j)),
            pl.BlockSpec((1, wt), lambda j: (0, j)),
        ],
        out_specs=[
            pl.BlockSpec((L, wt), lambda j: (0, j)),
            pl.BlockSpec((L, wt), lambda j: (0, j)),
            pl.BlockSpec((L, wt), lambda j: (0, j)),
        ],
        out_shape=[jax.ShapeDtypeStruct((L, D), f32)] * 3,
    )(x2, Wq, Wk, Wv, bq.reshape(1, D), bk.reshape(1, D), bv.reshape(1, D))

    m3 = pl.pallas_call(
        functools.partial(_mcol_kernel, seq=L, dh=dh, inv_sk=1.0 / sk),
        grid=(H // 2,),
        in_specs=[
            pl.BlockSpec((L, 2 * dh), lambda h: (0, h)),
            pl.BlockSpec((L, 2 * dh), lambda h: (0, h)),
            pl.BlockSpec((L, L), lambda h: (0, 0)),
        ],
        out_specs=pl.BlockSpec((1, L, 2), lambda h: (h, 0, 0)),
        out_shape=jax.ShapeDtypeStruct((H // 2, L, 2), f32),
        compiler_params=pltpu.CompilerParams(
            vmem_limit_bytes=100 * 1024 * 1024),
    )(q, k, counts)
    m_all = m3.reshape(H // 2, L, 2).transpose(1, 0, 2).reshape(L, H)

    ji = pl.pallas_call(
        functools.partial(_select_kernel, seq=L, n_heads=H, n_top=n_top),
        in_specs=[pl.BlockSpec((L, H), lambda: (0, 0))],
        out_specs=pl.BlockSpec((H, n_top), lambda: (0, 0)),
        out_shape=jax.ShapeDtypeStruct((H, n_top), jnp.int32),
    )(m_all)

    ctx = pl.pallas_call(
        functools.partial(_sattn_kernel, seq=L, dh=dh, n_top=n_top,
                          scale=1.0 / math.sqrt(dh)),
        grid=(H // 2,),
        in_specs=[
            pl.BlockSpec((L, 2 * dh), lambda h: (0, h)),
            pl.BlockSpec((L, 2 * dh), lambda h: (0, h)),
            pl.BlockSpec((L, 2 * dh), lambda h: (0, h)),
            pl.BlockSpec(memory_space=pltpu.SMEM),
        ],
        out_specs=pl.BlockSpec((L, 2 * dh), lambda h: (0, h)),
        out_shape=jax.ShapeDtypeStruct((L, D), f32),
        scratch_shapes=[
            pltpu.VMEM((n_top, dh), f32),
            pltpu.VMEM((n_top, dh), f32),
        ],
    )(q, k, v, ji)

    out = pl.pallas_call(
        _out_kernel,
        grid=(D // wt,),
        in_specs=[
            pl.BlockSpec((L, D), lambda j: (0, 0)),
            pl.BlockSpec((wt, D), lambda j: (j, 0)),
            pl.BlockSpec((1, wt), lambda j: (0, j)),
        ],
        out_specs=pl.BlockSpec((L, wt), lambda j: (0, j)),
        out_shape=jax.ShapeDtypeStruct((L, D), f32),
    )(ctx, Wo, bo.reshape(1, D))

    return out.reshape(B, L, D)
